# baseline (speedup 1.0000x reference)
import jax
import jax.numpy as jnp
from jax.experimental import pallas as pl
from jax.experimental.pallas import tpu as pltpu

H = 512
PATCH = 8
G = H // PATCH
N = G * G
D_IN = 3 * PATCH * PATCH
D = 256
NC = 80
NV = NC + 5
KEEP = 100
CONF_T = 0.05
IOU_T = 0.5

RB = 512
NBLK = N // RB


def _dense_body(x_ref, wbb_ref, bbb_ref, wh_ref, bh_ref,
                comb_ref, allp_ref):
    stripe = x_ref[...]
    x = stripe.reshape(3, 8, 8, 64, 8)
    x = jnp.transpose(x, (1, 3, 0, 2, 4)).reshape(RB, D_IN)
    feats = jnp.dot(x, wbb_ref[...], preferred_element_type=jnp.float32) + bbb_ref[...]
    feats = jnp.maximum(feats, 0.0)
    h = jnp.dot(feats, wh_ref[...], preferred_element_type=jnp.float32) + bh_ref[...]
    cls_probs = jax.nn.sigmoid(h[:, :NC])
    ctr_p = jax.nn.sigmoid(h[:, NC + 4:NC + 5])
    scores = cls_probs * ctr_p
    reg = h[:, NC:NC + 4]
    m = jnp.max(scores, axis=1, keepdims=True)
    cid = jax.lax.broadcasted_iota(jnp.int32, scores.shape, 1)
    lb = jnp.min(jnp.where(scores == m, cid, NC), axis=1, keepdims=True)
    lbf = lb.astype(jnp.float32)
    comb_ref[...] = jnp.concatenate([scores, reg, lbf], axis=1)
    packed = jnp.concatenate([reg, m, lbf], axis=1)
    allp_ref[...] = jnp.transpose(packed)[None]


def _nms_body(comb_ref, allp_ref, bo_ref, so_ref, lo_ref,
              km_ref, oh_ref):
    x1 = allp_ref[:, 0, :]
    y1 = allp_ref[:, 1, :]
    x2 = allp_ref[:, 2, :]
    y2 = allp_ref[:, 3, :]
    ms = allp_ref[:, 4, :]
    lbf = allp_ref[:, 5, :]
    mc = jnp.maximum(jnp.maximum(jnp.max(jnp.abs(x1)), jnp.max(jnp.abs(y1))),
                     jnp.maximum(jnp.max(jnp.abs(x2)), jnp.max(jnp.abs(y2)))) + 1.0
    off = lbf * mc
    ox1 = x1 + off
    oy1 = y1 + off
    ox2 = x2 + off
    oy2 = y2 + off
    a2 = jnp.maximum(ox2 - ox1, 0.0) * jnp.maximum(oy2 - oy1, 0.0)
    s = jnp.where(ms > CONF_T, ms, -1.0)

    def red2(a, op):
        return op(op(a, axis=0, keepdims=True), axis=1, keepdims=True)

    def body(k, s):
        m = red2(s, jnp.max)
        sel = s == m
        bx1 = red2(jnp.where(sel, ox1, 0.0), jnp.sum)
        by1 = red2(jnp.where(sel, oy1, 0.0), jnp.sum)
        bx2 = red2(jnp.where(sel, ox2, 0.0), jnp.sum)
        by2 = red2(jnp.where(sel, oy2, 0.0), jnp.sum)
        inter = (jnp.maximum(jnp.minimum(bx2, ox2) - jnp.maximum(bx1, ox1), 0.0)
                 * jnp.maximum(jnp.minimum(by2, oy2) - jnp.maximum(by1, oy1), 0.0))
        a1 = jnp.maximum(bx2 - bx1, 0.0) * jnp.maximum(by2 - by1, 0.0)
        iou = inter / (a1 + a2 - inter + 1e-6)
        km_ref[pl.ds(k, 1), :] = m
        oh_ref[pl.ds(k, 1), :, :] = sel.astype(jnp.float32)[None]
        return jnp.where((iou > IOU_T) | sel, -jnp.inf, s)

    s = jax.lax.fori_loop(0, KEEP, body, s)

    km = km_ref[...]
    valid = km > CONF_T
    vm = valid.astype(jnp.float32)
    acc = jnp.zeros((KEEP, NV), jnp.float32)
    for b in range(NBLK):
        acc = acc + jnp.dot(oh_ref[:, b, :], comb_ref[b * RB:(b + 1) * RB, :],
                            preferred_element_type=jnp.float32,
                            precision=jax.lax.Precision.HIGHEST)
    so_ref[...] = acc[:, :NC] * vm
    bo_ref[...] = acc[:, NC:NC + 4] * vm
    lo_ref[...] = jnp.where(valid, acc[:, NC + 4:NC + 5], -1.0).astype(jnp.int32)


def kernel(images, W_bb, b_bb, W_cls, b_cls, W_reg, b_reg, W_ctr, b_ctr):
    x = images.reshape(3, H, H)
    Wh = jnp.concatenate([W_cls, W_reg, W_ctr], axis=1)
    bh = jnp.concatenate([b_cls, b_reg, b_ctr])[None]
    bbb = b_bb[None]

    comb, allp = pl.pallas_call(
        _dense_body,
        grid=(NBLK,),
        in_specs=[
            pl.BlockSpec((3, G, H), lambda i: (0, i, 0)),
            pl.BlockSpec((D_IN, D), lambda i: (0, 0)),
            pl.BlockSpec((1, D), lambda i: (0, 0)),
            pl.BlockSpec((D, NV), lambda i: (0, 0)),
            pl.BlockSpec((1, NV), lambda i: (0, 0)),
        ],
        out_specs=[
            pl.BlockSpec((RB, NV), lambda i: (i, 0)),
            pl.BlockSpec((1, 6, RB), lambda i: (i, 0, 0)),
        ],
        out_shape=[
            jax.ShapeDtypeStruct((N, NV), jnp.float32),
            jax.ShapeDtypeStruct((NBLK, 6, RB), jnp.float32),
        ],
    )(x, W_bb, bbb, Wh, bh)

    bo, so, lo = pl.pallas_call(
        _nms_body,
        scratch_shapes=[
            pltpu.VMEM((KEEP, 1), jnp.float32),
            pltpu.VMEM((KEEP, NBLK, RB), jnp.float32),
        ],
        out_shape=[
            jax.ShapeDtypeStruct((KEEP, 4), jnp.float32),
            jax.ShapeDtypeStruct((KEEP, NC), jnp.float32),
            jax.ShapeDtypeStruct((KEEP, 1), jnp.int32),
        ],
    )(comb, allp)
    return bo, so, lo.reshape(KEEP)


# speedup vs baseline: 1.0137x; 1.0137x over previous
import jax
import jax.numpy as jnp
from jax.experimental import pallas as pl
from jax.experimental.pallas import tpu as pltpu

H = 512
PATCH = 8
G = H // PATCH
N = G * G
D_IN = 3 * PATCH * PATCH
D = 256
NC = 80
NV = NC + 5
KEEP = 100
CONF_T = 0.05
IOU_T = 0.5

RB = 512
NBLK = N // RB


def _dense_body(x_ref, wbb_ref, bbb_ref, wh_ref, bh_ref,
                comb_ref, allp_ref):
    stripe = x_ref[...]
    x = stripe.reshape(3, 8, 8, 64, 8)
    x = jnp.transpose(x, (1, 3, 0, 2, 4)).reshape(RB, D_IN)
    feats = jnp.dot(x, wbb_ref[...], preferred_element_type=jnp.float32) + bbb_ref[...]
    feats = jnp.maximum(feats, 0.0)
    h = jnp.dot(feats, wh_ref[...], preferred_element_type=jnp.float32) + bh_ref[...]
    cls_probs = jax.nn.sigmoid(h[:, :NC])
    ctr_p = jax.nn.sigmoid(h[:, NC + 4:NC + 5])
    scores = cls_probs * ctr_p
    reg = h[:, NC:NC + 4]
    m = jnp.max(scores, axis=1, keepdims=True)
    cid = jax.lax.broadcasted_iota(jnp.int32, scores.shape, 1)
    lb = jnp.min(jnp.where(scores == m, cid, NC), axis=1, keepdims=True)
    lbf = lb.astype(jnp.float32)
    comb_ref[...] = jnp.concatenate([scores, reg, lbf], axis=1)
    packed = jnp.concatenate([reg, m, lbf], axis=1)
    allp_ref[...] = jnp.transpose(packed)[None]


def _nms_body(comb_ref, allp_ref, bo_ref, so_ref, lo_ref,
              km_ref, oh_ref):
    x1 = allp_ref[:, 0, :]
    y1 = allp_ref[:, 1, :]
    x2 = allp_ref[:, 2, :]
    y2 = allp_ref[:, 3, :]
    ms = allp_ref[:, 4, :]
    lbf = allp_ref[:, 5, :]
    mc = jnp.maximum(jnp.maximum(jnp.max(jnp.abs(x1)), jnp.max(jnp.abs(y1))),
                     jnp.maximum(jnp.max(jnp.abs(x2)), jnp.max(jnp.abs(y2)))) + 1.0
    off = lbf * mc
    ox1 = x1 + off
    oy1 = y1 + off
    ox2 = x2 + off
    oy2 = y2 + off
    a2 = jnp.maximum(ox2 - ox1, 0.0) * jnp.maximum(oy2 - oy1, 0.0)
    s = jnp.where(ms > CONF_T, ms, -1.0)

    def red2(a, op):
        return op(op(a, axis=0, keepdims=True), axis=1, keepdims=True)

    for k in range(KEEP):
        m = red2(s, jnp.max)
        sel = s == m
        bx1 = red2(jnp.where(sel, ox1, 0.0), jnp.sum)
        by1 = red2(jnp.where(sel, oy1, 0.0), jnp.sum)
        bx2 = red2(jnp.where(sel, ox2, 0.0), jnp.sum)
        by2 = red2(jnp.where(sel, oy2, 0.0), jnp.sum)
        inter = (jnp.maximum(jnp.minimum(bx2, ox2) - jnp.maximum(bx1, ox1), 0.0)
                 * jnp.maximum(jnp.minimum(by2, oy2) - jnp.maximum(by1, oy1), 0.0))
        a1 = jnp.maximum(bx2 - bx1, 0.0) * jnp.maximum(by2 - by1, 0.0)
        iou = inter / (a1 + a2 - inter + 1e-6)
        km_ref[k:k + 1, :] = m
        oh_ref[k:k + 1, :, :] = sel.astype(jnp.float32)[None]
        s = jnp.where((iou > IOU_T) | sel, -jnp.inf, s)

    km = km_ref[...]
    valid = km > CONF_T
    vm = valid.astype(jnp.float32)
    acc = jnp.zeros((KEEP, NV), jnp.float32)
    for b in range(NBLK):
        acc = acc + jnp.dot(oh_ref[:, b, :], comb_ref[b * RB:(b + 1) * RB, :],
                            preferred_element_type=jnp.float32,
                            precision=jax.lax.Precision.HIGHEST)
    so_ref[...] = acc[:, :NC] * vm
    bo_ref[...] = acc[:, NC:NC + 4] * vm
    lo_ref[...] = jnp.where(valid, acc[:, NC + 4:NC + 5], -1.0).astype(jnp.int32)


def kernel(images, W_bb, b_bb, W_cls, b_cls, W_reg, b_reg, W_ctr, b_ctr):
    x = images.reshape(3, H, H)
    Wh = jnp.concatenate([W_cls, W_reg, W_ctr], axis=1)
    bh = jnp.concatenate([b_cls, b_reg, b_ctr])[None]
    bbb = b_bb[None]

    comb, allp = pl.pallas_call(
        _dense_body,
        grid=(NBLK,),
        in_specs=[
            pl.BlockSpec((3, G, H), lambda i: (0, i, 0)),
            pl.BlockSpec((D_IN, D), lambda i: (0, 0)),
            pl.BlockSpec((1, D), lambda i: (0, 0)),
            pl.BlockSpec((D, NV), lambda i: (0, 0)),
            pl.BlockSpec((1, NV), lambda i: (0, 0)),
        ],
        out_specs=[
            pl.BlockSpec((RB, NV), lambda i: (i, 0)),
            pl.BlockSpec((1, 6, RB), lambda i: (i, 0, 0)),
        ],
        out_shape=[
            jax.ShapeDtypeStruct((N, NV), jnp.float32),
            jax.ShapeDtypeStruct((NBLK, 6, RB), jnp.float32),
        ],
    )(x, W_bb, bbb, Wh, bh)

    bo, so, lo = pl.pallas_call(
        _nms_body,
        scratch_shapes=[
            pltpu.VMEM((KEEP, 1), jnp.float32),
            pltpu.VMEM((KEEP, NBLK, RB), jnp.float32),
        ],
        out_shape=[
            jax.ShapeDtypeStruct((KEEP, 4), jnp.float32),
            jax.ShapeDtypeStruct((KEEP, NC), jnp.float32),
            jax.ShapeDtypeStruct((KEEP, 1), jnp.int32),
        ],
    )(comb, allp)
    return bo, so, lo.reshape(KEEP)
